# baseline (speedup 1.0000x reference)
import functools

import jax
import jax.numpy as jnp
from jax import lax
from jax.experimental import pallas as pl
from jax.experimental.pallas import tpu as pltpu
from jax.experimental.pallas import tpu_sc as plsc

_TOK_BLOCK = 2048
_LANES = 16


def _tc_body(x_ref, wt_ref, b_ref, logits_ref, dl_ref, acc_ref, *,
             n_tokens, n_experts):
    g = pl.program_id(0)
    ng = pl.num_programs(0)

    lg = jnp.dot(x_ref[...], wt_ref[...],
                 preferred_element_type=jnp.float32)
    lt = lg.T + b_ref[...]
    logits_ref[...] = lt

    m1 = jnp.max(lt, axis=0, keepdims=True)
    e = jnp.exp(lt - m1)
    p = e / jnp.sum(e, axis=0, keepdims=True)
    psum = jnp.sum(p, axis=1, keepdims=True)

    @pl.when(g == 0)
    def _():
        acc_ref[...] = psum

    @pl.when(g != 0)
    def _():
        acc_ref[...] += psum

    @pl.when(g == ng - 1)
    def _():
        avg = acc_ref[...] / float(n_tokens)
        ent = -jnp.sum(avg * jnp.log(avg + 1e-8))
        max_ent = jnp.log(float(n_experts))
        dl_ref[...] = ((max_ent - ent) / max_ent).reshape(1, 1)


def _sc_topk_body(logits_hbm, w1_hbm, w2_hbm, i1_hbm, i2_hbm,
                  lv, w1v, w2v, i1v, i2v, *, chunk, n_experts, n_cores):
    wid = lax.axis_index("s") * n_cores + lax.axis_index("c")
    base = wid * chunk
    pltpu.sync_copy(logits_hbm.at[:, pl.ds(base, chunk)], lv)

    def per_group(g, _):
        sl = pl.ds(g * _LANES, _LANES)
        m1 = lv[0, sl]
        i1 = jnp.zeros((_LANES,), jnp.int32)
        m2 = jnp.full((_LANES,), -jnp.inf, jnp.float32)
        i2 = jnp.zeros((_LANES,), jnp.int32)

        def per_expert(e, carry):
            m1, i1, m2, i2 = carry
            ev = jnp.full((_LANES,), e, jnp.int32)
            v = lv[e, sl]
            gt = v > m1
            ge2 = v > m2
            m2n = jnp.where(gt, m1, jnp.where(ge2, v, m2))
            i2n = jnp.where(gt, i1, jnp.where(ge2, ev, i2))
            m1n = jnp.where(gt, v, m1)
            i1n = jnp.where(gt, ev, i1)
            return m1n, i1n, m2n, i2n

        m1, i1, m2, i2 = lax.fori_loop(1, n_experts, per_expert,
                                       (m1, i1, m2, i2), unroll=7)

        r = jnp.exp(m2 - m1)
        w1 = 1.0 / (1.0 + r)
        w1v[sl] = w1
        w2v[sl] = 1.0 - w1
        i1v[sl] = i1
        i2v[sl] = i2
        return ()

    lax.fori_loop(0, chunk // _LANES, per_group, ())

    pltpu.sync_copy(w1v, w1_hbm.at[pl.ds(base, chunk)])
    pltpu.sync_copy(w2v, w2_hbm.at[pl.ds(base, chunk)])
    pltpu.sync_copy(i1v, i1_hbm.at[pl.ds(base, chunk)])
    pltpu.sync_copy(i2v, i2_hbm.at[pl.ds(base, chunk)])


def kernel(x, W, b):
    B, S, H = x.shape
    E = W.shape[0]
    n_tokens = B * S
    tb = min(_TOK_BLOCK, n_tokens)
    ng = n_tokens // tb

    x2 = x.reshape(n_tokens, H)
    wt = W.T
    bc = b.reshape(E, 1)

    tc_body = functools.partial(_tc_body, n_tokens=n_tokens, n_experts=E)
    logits_t, dl = pl.pallas_call(
        tc_body,
        grid=(ng,),
        in_specs=[
            pl.BlockSpec((tb, H), lambda g: (g, 0)),
            pl.BlockSpec((H, E), lambda g: (0, 0)),
            pl.BlockSpec((E, 1), lambda g: (0, 0)),
        ],
        out_specs=[pl.BlockSpec((E, tb), lambda g: (0, g)),
                   pl.BlockSpec((1, 1), lambda g: (0, 0))],
        out_shape=[jax.ShapeDtypeStruct((E, n_tokens), jnp.float32),
                   jax.ShapeDtypeStruct((1, 1), jnp.float32)],
        scratch_shapes=[pltpu.VMEM((E, 1), jnp.float32)],
        compiler_params=pltpu.CompilerParams(
            dimension_semantics=("arbitrary",)),
    )(x2, wt, bc)

    info = plsc.get_sparse_core_info()
    nw = info.num_cores * info.num_subcores
    chunk = n_tokens // nw
    mesh = plsc.VectorSubcoreMesh(core_axis_name="c", subcore_axis_name="s")
    sc_body = functools.partial(_sc_topk_body, chunk=chunk, n_experts=E,
                                n_cores=info.num_cores)
    w1, w2, i1, i2 = pl.kernel(
        sc_body,
        out_type=[
            jax.ShapeDtypeStruct((n_tokens,), jnp.float32),
            jax.ShapeDtypeStruct((n_tokens,), jnp.float32),
            jax.ShapeDtypeStruct((n_tokens,), jnp.int32),
            jax.ShapeDtypeStruct((n_tokens,), jnp.int32),
        ],
        mesh=mesh,
        scratch_types=[
            pltpu.VMEM((E, chunk), jnp.float32),
            pltpu.VMEM((chunk,), jnp.float32),
            pltpu.VMEM((chunk,), jnp.float32),
            pltpu.VMEM((chunk,), jnp.int32),
            pltpu.VMEM((chunk,), jnp.int32),
        ],
    )(logits_t)

    routing_weights = jnp.stack([w1, w2], axis=-1).reshape(B, S, 2)
    selected_experts = jnp.stack([i1, i2], axis=-1).reshape(B, S, 2)
    return routing_weights, selected_experts, dl[0, 0]


# speedup vs baseline: 1.0356x; 1.0356x over previous
import functools

import jax
import jax.numpy as jnp
from jax import lax
from jax.experimental import pallas as pl
from jax.experimental.pallas import tpu as pltpu
from jax.experimental.pallas import tpu_sc as plsc

_TOK_BLOCK = 2048
_LANES = 16


def _tc_body(x_ref, w_ref, b_ref, logits_ref, dl_ref, acc_ref, *,
             n_tokens, n_experts):
    g = pl.program_id(0)
    ng = pl.num_programs(0)

    lt = lax.dot_general(w_ref[...], x_ref[...],
                         (((1,), (1,)), ((), ())),
                         preferred_element_type=jnp.float32) + b_ref[...]
    logits_ref[...] = lt

    m1 = jnp.max(lt, axis=0, keepdims=True)
    e = jnp.exp(lt - m1)
    p = e / jnp.sum(e, axis=0, keepdims=True)
    psum = jnp.sum(p, axis=1, keepdims=True)

    @pl.when(g == 0)
    def _():
        acc_ref[...] = psum

    @pl.when(g != 0)
    def _():
        acc_ref[...] += psum

    @pl.when(g == ng - 1)
    def _():
        avg = acc_ref[...] / float(n_tokens)
        ent = -jnp.sum(avg * jnp.log(avg + 1e-8))
        max_ent = jnp.log(float(n_experts))
        dl_ref[...] = ((max_ent - ent) / max_ent).reshape(1, 1)


def _sc_topk_body(logits_hbm, w1_hbm, w2_hbm, i1_hbm, i2_hbm,
                  lv, w1v, w2v, i1v, i2v, *, chunk, n_experts, n_cores):
    wid = lax.axis_index("s") * n_cores + lax.axis_index("c")
    base = wid * chunk
    pltpu.sync_copy(logits_hbm.at[:, pl.ds(base, chunk)], lv)

    def per_group(g, _):
        sl = pl.ds(g * _LANES, _LANES)
        m1 = lv[0, sl]
        i1 = jnp.zeros((_LANES,), jnp.int32)
        m2 = jnp.full((_LANES,), -jnp.inf, jnp.float32)
        i2 = jnp.zeros((_LANES,), jnp.int32)

        def per_expert(e, carry):
            m1, i1, m2, i2 = carry
            ev = jnp.full((_LANES,), e, jnp.int32)
            v = lv[e, sl]
            gt = v > m1
            ge2 = v > m2
            m2n = jnp.where(gt, m1, jnp.where(ge2, v, m2))
            i2n = jnp.where(gt, i1, jnp.where(ge2, ev, i2))
            m1n = jnp.where(gt, v, m1)
            i1n = jnp.where(gt, ev, i1)
            return m1n, i1n, m2n, i2n

        m1, i1, m2, i2 = lax.fori_loop(1, n_experts, per_expert,
                                       (m1, i1, m2, i2), unroll=7)

        r = jnp.exp(m2 - m1)
        w1 = 1.0 / (1.0 + r)
        w1v[sl] = w1
        w2v[sl] = 1.0 - w1
        i1v[sl] = i1
        i2v[sl] = i2
        return ()

    lax.fori_loop(0, chunk // _LANES, per_group, ())

    pltpu.sync_copy(w1v, w1_hbm.at[pl.ds(base, chunk)])
    pltpu.sync_copy(w2v, w2_hbm.at[pl.ds(base, chunk)])
    pltpu.sync_copy(i1v, i1_hbm.at[pl.ds(base, chunk)])
    pltpu.sync_copy(i2v, i2_hbm.at[pl.ds(base, chunk)])


def kernel(x, W, b):
    B, S, H = x.shape
    E = W.shape[0]
    n_tokens = B * S
    tb = min(_TOK_BLOCK, n_tokens)
    ng = n_tokens // tb

    x2 = x.reshape(n_tokens, H)
    bc = b.reshape(E, 1)

    tc_body = functools.partial(_tc_body, n_tokens=n_tokens, n_experts=E)
    logits_t, dl = pl.pallas_call(
        tc_body,
        grid=(ng,),
        in_specs=[
            pl.BlockSpec((tb, H), lambda g: (g, 0)),
            pl.BlockSpec((E, H), lambda g: (0, 0)),
            pl.BlockSpec((E, 1), lambda g: (0, 0)),
        ],
        out_specs=[pl.BlockSpec((E, tb), lambda g: (0, g)),
                   pl.BlockSpec((1, 1), lambda g: (0, 0))],
        out_shape=[jax.ShapeDtypeStruct((E, n_tokens), jnp.float32),
                   jax.ShapeDtypeStruct((1, 1), jnp.float32)],
        scratch_shapes=[pltpu.VMEM((E, 1), jnp.float32)],
        compiler_params=pltpu.CompilerParams(
            dimension_semantics=("arbitrary",)),
    )(x2, W, bc)

    info = plsc.get_sparse_core_info()
    nw = info.num_cores * info.num_subcores
    chunk = n_tokens // nw
    mesh = plsc.VectorSubcoreMesh(core_axis_name="c", subcore_axis_name="s")
    sc_body = functools.partial(_sc_topk_body, chunk=chunk, n_experts=E,
                                n_cores=info.num_cores)
    w1, w2, i1, i2 = pl.kernel(
        sc_body,
        out_type=[
            jax.ShapeDtypeStruct((n_tokens,), jnp.float32),
            jax.ShapeDtypeStruct((n_tokens,), jnp.float32),
            jax.ShapeDtypeStruct((n_tokens,), jnp.int32),
            jax.ShapeDtypeStruct((n_tokens,), jnp.int32),
        ],
        mesh=mesh,
        scratch_types=[
            pltpu.VMEM((E, chunk), jnp.float32),
            pltpu.VMEM((chunk,), jnp.float32),
            pltpu.VMEM((chunk,), jnp.float32),
            pltpu.VMEM((chunk,), jnp.int32),
            pltpu.VMEM((chunk,), jnp.int32),
        ],
    )(logits_t)

    routing_weights = jnp.stack([w1, w2], axis=-1).reshape(B, S, 2)
    selected_experts = jnp.stack([i1, i2], axis=-1).reshape(B, S, 2)
    return routing_weights, selected_experts, dl[0, 0]
